# Initial kernel scaffold; baseline (speedup 1.0000x reference)
import jax
import jax.numpy as jnp
from jax.experimental import pallas as pl


def kernel(item_ids, table):
    raise NotImplementedError("write your pallas kernel here")


# speedup vs baseline: 2.9631x; 2.9631x over previous
import functools
import jax
import jax.numpy as jnp
from jax import lax
from jax.experimental import pallas as pl
from jax.experimental.pallas import tpu as pltpu
from jax.experimental.pallas import tpu_sc as plsc

D_MODEL = 128
N_IDX = 4096 * 50
NUM_CORES = 2
NUM_SUBCORES = 16
NW = NUM_CORES * NUM_SUBCORES
PER_W = N_IDX // NW
CHUNK = 128
NCH = PER_W // CHUNK


def _emb_body(table_hbm, idx_hbm, out_hbm, idx_v, rows_v, sem):
    wid = lax.axis_index("s") * NUM_CORES + lax.axis_index("c")
    base = wid * PER_W
    pltpu.sync_copy(idx_hbm.at[wid], idx_v)

    def chunk(j, _):
        pltpu.async_copy(table_hbm.at[idx_v.at[j]], rows_v, sem).wait()
        pltpu.sync_copy(rows_v, out_hbm.at[pl.ds(base + j * CHUNK, CHUNK)])
        return 0

    lax.fori_loop(0, NCH, chunk, 0)


@jax.jit
def _emb_call(table, idx2d):
    mesh = plsc.VectorSubcoreMesh(core_axis_name="c", subcore_axis_name="s")
    return pl.kernel(
        _emb_body,
        out_type=jax.ShapeDtypeStruct((N_IDX, D_MODEL), jnp.float32),
        mesh=mesh,
        scratch_types=[
            pltpu.VMEM((NCH, CHUNK), jnp.int32),
            pltpu.VMEM((CHUNK, D_MODEL), jnp.float32),
            pltpu.SemaphoreType.DMA,
        ],
    )(table, idx2d)


def kernel(item_ids, table):
    idx2d = item_ids.astype(jnp.int32).reshape(NW, NCH, CHUNK)
    out = _emb_call(table, idx2d)
    return out.reshape(item_ids.shape + (D_MODEL,))


# speedup vs baseline: 3.3476x; 1.1297x over previous
import functools
import jax
import jax.numpy as jnp
from jax import lax
from jax.experimental import pallas as pl
from jax.experimental.pallas import tpu as pltpu
from jax.experimental.pallas import tpu_sc as plsc

D_MODEL = 128
N_IDX = 4096 * 50
NUM_CORES = 2
NUM_SUBCORES = 16
NW = NUM_CORES * NUM_SUBCORES
PER_W = N_IDX // NW
CHUNK = 128
NCH = PER_W // CHUNK


NBUF = 5


def _emb_body(table_hbm, idx_hbm, out_hbm, idx_v, rows, gsems, wsems):
    wid = lax.axis_index("s") * NUM_CORES + lax.axis_index("c")
    base = wid * PER_W
    pltpu.sync_copy(idx_hbm.at[wid], idx_v)

    def start_gather(j, b):
        pltpu.make_async_copy(table_hbm.at[idx_v.at[j]], rows[b], gsems[b]).start()

    def wait_gather(b):
        pltpu.make_async_copy(table_hbm.at[idx_v.at[0]], rows[b], gsems[b]).wait()

    def start_write(j, b):
        pltpu.make_async_copy(
            rows[b], out_hbm.at[pl.ds(base + j * CHUNK, CHUNK)], wsems[b]
        ).start()

    def wait_write(b):
        pltpu.make_async_copy(
            rows[b], out_hbm.at[pl.ds(base, CHUNK)], wsems[b]
        ).wait()

    for b in range(NBUF):
        start_gather(b, b)

    def step(i, _):
        j0 = i * NBUF
        for b in range(NBUF):
            j = j0 + b
            wait_gather(b)
            start_write(j, b)

            @pl.when(j + NBUF < NCH)
            def _():
                wait_write(b)
                start_gather(j + NBUF, b)

        return 0

    lax.fori_loop(0, NCH // NBUF, step, 0)
    for b in range(NBUF):
        wait_write(b)


@jax.jit
def _emb_call(table, idx2d):
    mesh = plsc.VectorSubcoreMesh(core_axis_name="c", subcore_axis_name="s")
    return pl.kernel(
        _emb_body,
        out_type=jax.ShapeDtypeStruct((N_IDX, D_MODEL), jnp.float32),
        mesh=mesh,
        scratch_types=[
            pltpu.VMEM((NCH, CHUNK), jnp.int32),
            [pltpu.VMEM((CHUNK, D_MODEL), jnp.float32) for _ in range(NBUF)],
            [pltpu.SemaphoreType.DMA for _ in range(NBUF)],
            [pltpu.SemaphoreType.DMA for _ in range(NBUF)],
        ],
    )(table, idx2d)


def kernel(item_ids, table):
    idx2d = item_ids.astype(jnp.int32).reshape(NW, NCH, CHUNK)
    out = _emb_call(table, idx2d)
    return out.reshape(item_ids.shape + (D_MODEL,))


# speedup vs baseline: 5.9636x; 1.7815x over previous
import functools
import jax
import jax.numpy as jnp
from jax import lax
from jax.experimental import pallas as pl
from jax.experimental.pallas import tpu as pltpu
from jax.experimental.pallas import tpu_sc as plsc

BATCH = 4096
SEQ = 50
D_MODEL = 128
NUM_CORES = 2
NUM_SUBCORES = 16
NW = NUM_CORES * NUM_SUBCORES
BPW = BATCH // NW
NB = 4
NCH = BPW // NB
NBUF = 4


def _emb_body(table_hbm, idx_hbm, out_hbm, idx_v, bufs, gsems, wsems):
    wid = lax.axis_index("s") * NUM_CORES + lax.axis_index("c")
    pltpu.sync_copy(idx_hbm.at[wid], idx_v)

    def start_gathers(c, r):
        for k in range(NB):
            pltpu.make_async_copy(
                table_hbm.at[idx_v.at[c * NB + k]], bufs[r].at[k], gsems[r]
            ).start()

    def wait_gathers(r):
        for k in range(NB):
            pltpu.make_async_copy(
                table_hbm.at[idx_v.at[0]], bufs[r].at[k], gsems[r]
            ).wait()

    def start_write(c, r):
        pltpu.make_async_copy(
            bufs[r], out_hbm.at[pl.ds(wid * BPW + c * NB, NB)], wsems[r]
        ).start()

    def wait_write(r):
        pltpu.make_async_copy(
            bufs[r], out_hbm.at[pl.ds(wid * BPW, NB)], wsems[r]
        ).wait()

    for r in range(NBUF):
        start_gathers(r, r)

    def step(i, _):
        c0 = i * NBUF
        for r in range(NBUF):
            c = c0 + r
            wait_gathers(r)
            start_write(c, r)

            @pl.when(c + NBUF < NCH)
            def _():
                wait_write(r)
                start_gathers(c + NBUF, r)

        return 0

    lax.fori_loop(0, NCH // NBUF, step, 0)
    for r in range(NBUF):
        wait_write(r)


@jax.jit
def _emb_call(table, idx3):
    mesh = plsc.VectorSubcoreMesh(core_axis_name="c", subcore_axis_name="s")
    return pl.kernel(
        _emb_body,
        out_type=jax.ShapeDtypeStruct((BATCH, SEQ, D_MODEL), jnp.float32),
        mesh=mesh,
        scratch_types=[
            pltpu.VMEM((BPW, SEQ), jnp.int32),
            [pltpu.VMEM((NB, SEQ, D_MODEL), jnp.float32) for _ in range(NBUF)],
            [pltpu.SemaphoreType.DMA for _ in range(NBUF)],
            [pltpu.SemaphoreType.DMA for _ in range(NBUF)],
        ],
    )(table, idx3)


def kernel(item_ids, table):
    idx3 = item_ids.astype(jnp.int32).reshape(NW, BPW, SEQ)
    return _emb_call(table, idx3)


# speedup vs baseline: 10.4355x; 1.7498x over previous
import functools
import jax
import jax.numpy as jnp
from jax import lax
from jax.experimental import pallas as pl
from jax.experimental.pallas import tpu as pltpu
from jax.experimental.pallas import tpu_sc as plsc

BATCH = 4096
SEQ = 50
D_MODEL = 128
N_IDX = BATCH * SEQ
NUM_CORES = 2
NUM_SUBCORES = 16
NW = NUM_CORES * NUM_SUBCORES
PER_W = N_IDX // NW
CHUNK = 128
NCH = PER_W // CHUNK
NBUF = 5


def _emb_body(table_hbm, idx_hbm, out_hbm, idx_v, rows, gsems, wsems):
    wid = lax.axis_index("s") * NUM_CORES + lax.axis_index("c")
    base = wid * PER_W
    pltpu.sync_copy(idx_hbm.at[wid], idx_v)

    def start_gather(j, b):
        pltpu.make_async_copy(table_hbm.at[idx_v.at[j]], rows[b], gsems[b]).start()

    def wait_gather(b):
        pltpu.make_async_copy(table_hbm.at[idx_v.at[0]], rows[b], gsems[b]).wait()

    def start_write(j, b):
        pltpu.make_async_copy(
            rows[b], out_hbm.at[pl.ds(base + j * CHUNK, CHUNK)], wsems[b]
        ).start()

    def wait_write(b):
        pltpu.make_async_copy(
            rows[b], out_hbm.at[pl.ds(base, CHUNK)], wsems[b]
        ).wait()

    for b in range(NBUF):
        start_gather(b, b)

    def step(i, _):
        j0 = i * NBUF
        for b in range(NBUF):
            j = j0 + b
            wait_gather(b)
            start_write(j, b)

            @pl.when(j + NBUF < NCH)
            def _():
                wait_write(b)
                start_gather(j + NBUF, b)

        return 0

    lax.fori_loop(0, NCH // NBUF, step, 0)
    for b in range(NBUF):
        wait_write(b)


@jax.jit
def _emb_call(table, idx3):
    mesh = plsc.VectorSubcoreMesh(core_axis_name="c", subcore_axis_name="s")
    out = pl.kernel(
        _emb_body,
        out_type=jax.ShapeDtypeStruct((N_IDX, D_MODEL), jnp.float32),
        mesh=mesh,
        scratch_types=[
            pltpu.VMEM((NCH, CHUNK), jnp.int32),
            [pltpu.VMEM((CHUNK, D_MODEL), jnp.float32) for _ in range(NBUF)],
            [pltpu.SemaphoreType.DMA for _ in range(NBUF)],
            [pltpu.SemaphoreType.DMA for _ in range(NBUF)],
        ],
    )(table, idx3)
    return out.reshape(SEQ, BATCH, D_MODEL).transpose(1, 0, 2)


def kernel(item_ids, table):
    idx3 = jnp.transpose(item_ids).astype(jnp.int32).reshape(NW, NCH, CHUNK)
    return _emb_call(table, idx3)


# speedup vs baseline: 10.4672x; 1.0030x over previous
import functools
import jax
import jax.numpy as jnp
from jax import lax
from jax.experimental import pallas as pl
from jax.experimental.pallas import tpu as pltpu
from jax.experimental.pallas import tpu_sc as plsc

BATCH = 4096
SEQ = 50
D_MODEL = 128
N_IDX = BATCH * SEQ
NUM_CORES = 2
NUM_SUBCORES = 16
NW = NUM_CORES * NUM_SUBCORES
PER_W = N_IDX // NW
CHUNK = 64
NCH = PER_W // CHUNK
NBUF = 10


def _emb_body(table_hbm, idx_hbm, out_hbm, idx_v, rows, gsems, wsems):
    wid = lax.axis_index("s") * NUM_CORES + lax.axis_index("c")
    base = wid * PER_W
    pltpu.sync_copy(idx_hbm.at[wid], idx_v)

    def start_gather(j, b):
        pltpu.make_async_copy(table_hbm.at[idx_v.at[j]], rows[b], gsems[b]).start()

    def wait_gather(b):
        pltpu.make_async_copy(table_hbm.at[idx_v.at[0]], rows[b], gsems[b]).wait()

    def start_write(j, b):
        pltpu.make_async_copy(
            rows[b], out_hbm.at[pl.ds(base + j * CHUNK, CHUNK)], wsems[b]
        ).start()

    def wait_write(b):
        pltpu.make_async_copy(
            rows[b], out_hbm.at[pl.ds(base, CHUNK)], wsems[b]
        ).wait()

    for b in range(NBUF):
        start_gather(b, b)

    def step(i, _):
        j0 = i * NBUF
        for b in range(NBUF):
            j = j0 + b
            wait_gather(b)
            start_write(j, b)

            @pl.when(j + NBUF < NCH)
            def _():
                wait_write(b)
                start_gather(j + NBUF, b)

        return 0

    lax.fori_loop(0, NCH // NBUF, step, 0)
    for b in range(NBUF):
        wait_write(b)


@jax.jit
def _emb_call(table, idx3):
    mesh = plsc.VectorSubcoreMesh(core_axis_name="c", subcore_axis_name="s")
    out = pl.kernel(
        _emb_body,
        out_type=jax.ShapeDtypeStruct((N_IDX, D_MODEL), jnp.float32),
        mesh=mesh,
        scratch_types=[
            pltpu.VMEM((NCH, CHUNK), jnp.int32),
            [pltpu.VMEM((CHUNK, D_MODEL), jnp.float32) for _ in range(NBUF)],
            [pltpu.SemaphoreType.DMA for _ in range(NBUF)],
            [pltpu.SemaphoreType.DMA for _ in range(NBUF)],
        ],
    )(table, idx3)
    return out.reshape(SEQ, BATCH, D_MODEL).transpose(1, 0, 2)


def kernel(item_ids, table):
    idx3 = jnp.transpose(item_ids).astype(jnp.int32).reshape(NW, NCH, CHUNK)
    return _emb_call(table, idx3)
